# Initial kernel scaffold; baseline (speedup 1.0000x reference)
import jax
import jax.numpy as jnp
from jax.experimental import pallas as pl


def kernel(inputs, embedding):
    raise NotImplementedError("write your pallas kernel here")


# speedup vs baseline: 1.6832x; 1.6832x over previous
import functools

import jax
import jax.numpy as jnp
from jax import lax
from jax.experimental import pallas as pl
from jax.experimental.pallas import tpu as pltpu
from jax.experimental.pallas import tpu_sc as plsc

_D = 64
_NC = 2
_NS = 16
_NW = _NC * _NS
_CHUNK = 128


def _gather_body(table_hbm, idx_hbm, out_hbm, idx_v, rows_v, sem):
    c = lax.axis_index("c")
    s = lax.axis_index("s")
    wid = s * _NC + c
    n_chunks = idx_v.shape[0]
    n_per_w = n_chunks * _CHUNK
    base = wid * n_per_w

    pltpu.sync_copy(idx_hbm.at[wid], idx_v)

    def chunk(j, carry):
        pltpu.async_copy(table_hbm.at[idx_v.at[j]], rows_v, sem).wait()
        pltpu.sync_copy(rows_v, out_hbm.at[pl.ds(base + j * _CHUNK, _CHUNK)])
        return carry

    lax.fori_loop(0, n_chunks, chunk, 0)


def _make_gather(n_flat: int):
    n_per_w = n_flat // _NW
    n_chunks = n_per_w // _CHUNK
    mesh = plsc.VectorSubcoreMesh(
        core_axis_name="c", subcore_axis_name="s",
        num_cores=_NC, num_subcores=_NS)
    return pl.kernel(
        _gather_body,
        out_type=jax.ShapeDtypeStruct((n_flat, _D), jnp.float32),
        mesh=mesh,
        scratch_types=[
            pltpu.VMEM((n_chunks, _CHUNK), jnp.int32),
            pltpu.VMEM((_CHUNK, _D), jnp.float32),
            pltpu.SemaphoreType.DMA,
        ],
        compiler_params=pltpu.CompilerParams(use_tc_tiling_on_sc=False),
    )


@jax.jit
def kernel(inputs, embedding):
    b, h = inputs.shape
    n_flat = b * h
    idx = inputs.reshape(_NW, n_flat // (_NW * _CHUNK), _CHUNK).astype(jnp.int32)
    out = _make_gather(n_flat)(embedding, idx)
    return out.reshape(b, h, _D)


# speedup vs baseline: 1.8765x; 1.1149x over previous
import functools

import jax
import jax.numpy as jnp
from jax import lax
from jax.experimental import pallas as pl
from jax.experimental.pallas import tpu as pltpu
from jax.experimental.pallas import tpu_sc as plsc

_D = 64
_NC = 2
_NS = 16
_NW = _NC * _NS
_CHUNK = 128
_NBUF = 4


def _gather_body(table_hbm, idx_hbm, out_hbm, idx_v, rows, sems):
    c = lax.axis_index("c")
    s = lax.axis_index("s")
    wid = s * _NC + c
    n_chunks = idx_v.shape[0]
    n_per_w = n_chunks * _CHUNK
    base = wid * n_per_w

    pltpu.sync_copy(idx_hbm.at[wid], idx_v)

    def start_gather(j, b):
        pltpu.async_copy(table_hbm.at[idx_v.at[j]], rows[b], sems[b])

    def wait_gather(j, b):
        pltpu.make_async_copy(table_hbm.at[idx_v.at[j]], rows[b], sems[b]).wait()

    def store(j, b):
        pltpu.sync_copy(rows[b], out_hbm.at[pl.ds(base + j * _CHUNK, _CHUNK)])

    for b in range(_NBUF):
        start_gather(b, b)

    n_groups = n_chunks // _NBUF - 1

    def group(gi, carry):
        j0 = gi * _NBUF
        for b in range(_NBUF):
            wait_gather(j0 + b, b)
            store(j0 + b, b)
            start_gather(j0 + _NBUF + b, b)
        return carry

    lax.fori_loop(0, n_groups, group, 0)

    j0 = n_groups * _NBUF
    for b in range(_NBUF):
        wait_gather(j0 + b, b)
        store(j0 + b, b)


def _make_gather(n_flat: int):
    n_per_w = n_flat // _NW
    n_chunks = n_per_w // _CHUNK
    mesh = plsc.VectorSubcoreMesh(
        core_axis_name="c", subcore_axis_name="s",
        num_cores=_NC, num_subcores=_NS)
    return pl.kernel(
        _gather_body,
        out_type=jax.ShapeDtypeStruct((n_flat, _D), jnp.float32),
        mesh=mesh,
        scratch_types=[
            pltpu.VMEM((n_chunks, _CHUNK), jnp.int32),
            [pltpu.VMEM((_CHUNK, _D), jnp.float32) for _ in range(_NBUF)],
            [pltpu.SemaphoreType.DMA for _ in range(_NBUF)],
        ],
        compiler_params=pltpu.CompilerParams(use_tc_tiling_on_sc=False),
    )


@jax.jit
def kernel(inputs, embedding):
    b, h = inputs.shape
    n_flat = b * h
    idx = inputs.reshape(_NW, n_flat // (_NW * _CHUNK), _CHUNK).astype(jnp.int32)
    out = _make_gather(n_flat)(embedding, idx)
    return out.reshape(b, h, _D)
